# baseline (speedup 1.0000x reference)
import functools
import math

import jax
import jax.numpy as jnp
from jax import lax
from jax.experimental import pallas as pl
from jax.experimental.pallas import tpu as pltpu
from jax.experimental.pallas import tpu_sc as plsc

MODEL_DIM = 64
VOCAB = 1000000
N_PAIR = VOCAB // 2
SCALE = math.sqrt(MODEL_DIM)

NUM_CORES = 2
NUM_SUBCORES = 16
NUM_WORKERS = NUM_CORES * NUM_SUBCORES
LANES = 16

TT = 8
IB = 128

_MESH = dict(core_axis_name="c", subcore_axis_name="s",
             num_cores=NUM_CORES, num_subcores=NUM_SUBCORES)

TC_W = 4096
N_GRID = 123
H = N_GRID * TC_W


def _make_pair_kernel():
    def body(lo_ref, hi_ref, out_ref):
        out_ref[:, 0:MODEL_DIM] = lo_ref[...].T
        out_ref[:, MODEL_DIM:2 * MODEL_DIM] = hi_ref[...].T

    return pl.pallas_call(
        body,
        grid=(N_GRID,),
        in_specs=[
            pl.BlockSpec((MODEL_DIM, TC_W), lambda j: (0, j)),
            pl.BlockSpec((MODEL_DIM, TC_W),
                         lambda j: (0, jnp.minimum(j + N_GRID,
                                                   VOCAB // TC_W))),
        ],
        out_specs=pl.BlockSpec((TC_W, 2 * MODEL_DIM), lambda j: (j, 0)),
        out_shape=jax.ShapeDtypeStruct((H, 2 * MODEL_DIM), jnp.float32),
    )


def _make_lookup_kernel(T: int, N: int):
    n_ib = N // IB
    units = (T // TT) * n_ib
    units_w = units // NUM_WORKERS

    @functools.partial(
        pl.kernel,
        out_type=jax.ShapeDtypeStruct((T, MODEL_DIM, N), jnp.float32),
        mesh=plsc.VectorSubcoreMesh(**_MESH),
        scratch_types=[
            pltpu.VMEM((TT, IB), jnp.int32),
            pltpu.VMEM((TT, IB), jnp.int32),
            pltpu.VMEM((TT, IB), jnp.int32),
            pltpu.VMEM((IB, 2 * MODEL_DIM), jnp.float32),
            pltpu.VMEM((IB, 2 * MODEL_DIM), jnp.float32),
            pltpu.VMEM((IB, 2 * MODEL_DIM), jnp.float32),
            pltpu.VMEM((IB, 2 * MODEL_DIM), jnp.float32),
            pltpu.VMEM((MODEL_DIM, IB), jnp.float32),
            pltpu.VMEM((MODEL_DIM, IB), jnp.float32),
            pltpu.SemaphoreType.DMA,
            pltpu.SemaphoreType.DMA,
        ],
        compiler_params=pltpu.CompilerParams(use_tc_tiling_on_sc=True,
                                             needs_layout_passes=False),
    )
    def lkp(xt_hbm, pairs_hbm, out_hbm, idxt_v, pidx_v, hv_v,
            prows_a, prows_b, prows_c, prows_d, trans_a, trans_b,
            gsem, wsem):
        wid = lax.axis_index("s") * NUM_CORES + lax.axis_index("c")
        lane = lax.iota(jnp.int32, LANES)
        prows = (prows_a, prows_b, prows_c, prows_d)
        trans = (trans_a, trans_b)
        half = IB // 2

        def fire(t8):
            buf = prows[t8 & 3]
            return [
                pltpu.async_copy(
                    pairs_hbm.at[pidx_v.at[t8, pl.ds(hb * half, half)]],
                    buf.at[pl.ds(hb * half, half), :], gsem)
                for hb in range(2)
            ]

        def unit(k, _):
            u = wid + NUM_WORKERS * k
            tb = u // n_ib
            ib = u % n_ib
            pltpu.sync_copy(
                xt_hbm.at[pl.ds(tb * TT, TT), pl.ds(ib * IB, IB)], idxt_v)

            @plsc.parallel_loop(0, TT * (IB // LANES), unroll=8)
            def mkpidx(q):
                r = q // (IB // LANES)
                c0 = (q % (IB // LANES)) * LANES
                v = idxt_v[r, pl.ds(c0, LANES)]
                hi = v >= H
                pidx_v[r, pl.ds(c0, LANES)] = v - jnp.where(hi, H, 0)
                hv_v[r, pl.ds(c0, LANES)] = jnp.where(hi, MODEL_DIM, 0)

            gets = [None] * TT
            puts = [None] * TT
            for t8 in range(3):
                gets[t8] = fire(t8)
            for t8 in range(TT):
                for c in gets[t8]:
                    c.wait()
                if t8 + 3 < TT:
                    gets[t8 + 3] = fire(t8 + 3)
                if t8 >= 2:
                    puts[t8 - 2].wait()
                src = prows[t8 & 3]
                dst = trans[t8 & 1]

                for gi in range(IB // LANES):
                    hv16 = hv_v[t8, pl.ds(gi * LANES, LANES)]
                    rowv = lane + gi * LANES

                    @plsc.parallel_loop(0, MODEL_DIM, unroll=16)
                    def col(c):
                        val = plsc.load_gather(src, [rowv, hv16 + c])
                        dst[c, pl.ds(gi * LANES, LANES)] = val * SCALE

                puts[t8] = pltpu.async_copy(
                    dst, out_hbm.at[tb * TT + t8, :, pl.ds(ib * IB, IB)],
                    wsem)
            puts[TT - 2].wait()
            puts[TT - 1].wait()
            return 0

        lax.fori_loop(0, units_w, unit, 0)

    return lkp


def kernel(x, table):
    T, N = x.shape[1], x.shape[0]
    xt = x.T.astype(jnp.int32)
    tt = table.T
    pairs = _make_pair_kernel()(tt, tt)
    out_t = _make_lookup_kernel(T, N)(xt, pairs)
    return jnp.transpose(out_t, (2, 0, 1))
